# baseline (speedup 1.0000x reference)
import functools

import jax
import jax.numpy as jnp
from jax import lax
from jax.experimental import pallas as pl
from jax.experimental.pallas import tpu as pltpu
from jax.experimental.pallas import tpu_sc as plsc

D = 512
K = 16
NC, NS = 2, 16
NW = NC * NS
CH0 = 288
CH1 = 352
CHMAX = max(CH0, CH1)
BP = NS * (CH0 + CH1)
OG = 32
GE = 4
NBUF = 2
LANES = 16

_mesh = plsc.VectorSubcoreMesh(
    core_axis_name="c", subcore_axis_name="s", num_cores=NC, num_subcores=NS)


@functools.partial(
    pl.kernel,
    out_type=jax.ShapeDtypeStruct((BP, D), jnp.float32),
    mesh=_mesh,
    scratch_types=[
        pltpu.VMEM((CHMAX * K,), jnp.int32),
        pltpu.VMEM((NBUF, GE * K, D), jnp.float32),
        pltpu.VMEM((OG, D), jnp.float32),
        pltpu.VMEM((8, D), jnp.float32),
        pltpu.SemaphoreType.DMA,
        pltpu.SemaphoreType.DMA,
    ],
)
def _gather_mean(table, idx, out, idx_v, rows_v, out_v, x0_v, *gsems):
    c = lax.axis_index("c")
    sid = lax.axis_index("s")
    base = pl.multiple_of(sid * (CH0 + CH1) + c * CH0, OG)
    ngrp = jnp.where(c == 0, CH0 // GE, CH1 // GE)
    pltpu.sync_copy(
        idx.at[pl.ds(pl.multiple_of(base * K, 8), CHMAX * K)], idx_v)
    pltpu.sync_copy(table.at[pl.ds(0, 8)], x0_v)

    def issue(g, b):
        gofs = pl.multiple_of(g * (GE * K), 8)
        pltpu.async_copy(
            table.at[idx_v.at[pl.ds(gofs, GE * K)]], rows_v.at[b], gsems[b])

    for b0 in range(NBUF):
        issue(b0, b0)

    lanes = lax.iota(jnp.int32, LANES)
    gdn = lax.GatherDimensionNumbers(
        offset_dims=(), collapsed_slice_dims=(0,), start_index_map=(0,))

    def ring_body(i, carry):
        for b in range(NBUF):
            g = i * NBUF + b
            gofs = pl.multiple_of(g * (GE * K), 8)
            pltpu.make_async_copy(
                table.at[idx_v.at[pl.ds(gofs, GE * K)]], rows_v.at[b],
                gsems[b]).wait()
            for ee in range(GE):
                e = g * GE + ee
                iv = idx_v[pl.ds((g * GE + ee) * K, K)]
                cnt = jnp.minimum(iv.astype(jnp.float32), 1.0)
                for s in (1, 2, 4, 8):
                    perm = jnp.bitwise_xor(lanes, s)
                    shuf = lax.gather(cnt, perm[:, None], gdn, (1,),
                                      mode=lax.GatherScatterMode.PROMISE_IN_BOUNDS)
                    cnt = cnt + shuf
                cnt1 = jnp.maximum(cnt, 1.0)
                scale = 1.0 / cnt1
                c0 = 16.0 - cnt1
                slot = lax.rem(e, OG)

                def fbody(f, fc):
                    col = pl.ds(f * LANES, LANES)
                    acc = rows_v[b, ee * K, col]
                    for k2 in range(1, K):
                        acc = acc + rows_v[b, ee * K + k2, col]
                    out_v[slot, col] = (acc - c0 * x0_v[0, col]) * scale
                    return fc

                lax.fori_loop(0, D // LANES, fbody, 0, unroll=2)

                @pl.when(slot == OG - 1)
                def _():
                    row0 = pl.multiple_of(base + e - (OG - 1), OG)
                    pltpu.sync_copy(out_v, out.at[pl.ds(row0, OG)])

            @pl.when(g + NBUF < ngrp)
            def _():
                issue(g + NBUF, b)
        return carry

    lax.fori_loop(0, ngrp // NBUF, ring_body, 0)


def _mm_body(a_ref, w1_ref, w2_ref, o_ref):
    t = jnp.dot(a_ref[...], w1_ref[...], preferred_element_type=jnp.float32)
    t = jnp.maximum(t, 0.0)
    o_ref[...] = jnp.dot(t, w2_ref[...], preferred_element_type=jnp.float32)


def _mm(a, W1, W2):
    br = 512
    return pl.pallas_call(
        _mm_body,
        grid=(BP // br,),
        in_specs=[
            pl.BlockSpec((br, D), lambda i: (i, 0)),
            pl.BlockSpec((D, D), lambda i: (0, 0)),
            pl.BlockSpec((D, D), lambda i: (0, 0)),
        ],
        out_specs=pl.BlockSpec((br, D), lambda i: (i, 0)),
        out_shape=jax.ShapeDtypeStruct((BP, D), jnp.float32),
    )(a, W1, W2)


def kernel(x, seq, useq, W1, W2, att_w, att_b):
    e_rows = seq.shape[0]
    m_rows = useq.shape[0]
    seq_p = jnp.pad(seq, ((0, BP - e_rows), (0, 0))).reshape(-1)
    useq_p = jnp.pad(useq, ((0, BP - m_rows), (0, 0))).reshape(-1)
    agg1 = _gather_mean(x, seq_p)
    h = _mm(agg1, W1, W2)
    node = _gather_mean(h, useq_p)
    return node[:m_rows]


# speedup vs baseline: 1.1353x; 1.1353x over previous
import functools

import jax
import jax.numpy as jnp
from jax import lax
from jax.experimental import pallas as pl
from jax.experimental.pallas import tpu as pltpu
from jax.experimental.pallas import tpu_sc as plsc

D = 512
K = 16
NC, NS = 2, 16
NW = NC * NS
CH0 = 352
CH1 = 288
CHMAX = max(CH0, CH1)
BP = NS * (CH0 + CH1)
OG = 32
GE = 4
NBUF = 2
LANES = 16

_mesh = plsc.VectorSubcoreMesh(
    core_axis_name="c", subcore_axis_name="s", num_cores=NC, num_subcores=NS)


@functools.partial(
    pl.kernel,
    out_type=jax.ShapeDtypeStruct((BP, D), jnp.float32),
    mesh=_mesh,
    scratch_types=[
        pltpu.VMEM((CHMAX * K,), jnp.int32),
        pltpu.VMEM((NBUF, GE * K, D), jnp.float32),
        pltpu.VMEM((OG, D), jnp.float32),
        pltpu.VMEM((8, D), jnp.float32),
        pltpu.SemaphoreType.DMA,
        pltpu.SemaphoreType.DMA,
    ],
)
def _gather_mean(table, idx, out, idx_v, rows_v, out_v, x0_v, *gsems):
    c = lax.axis_index("c")
    sid = lax.axis_index("s")
    base = pl.multiple_of(sid * (CH0 + CH1) + c * CH0, OG)
    ngrp = jnp.where(c == 0, CH0 // GE, CH1 // GE)
    pltpu.sync_copy(
        idx.at[pl.ds(pl.multiple_of(base * K, 8), CHMAX * K)], idx_v)
    pltpu.sync_copy(table.at[pl.ds(0, 8)], x0_v)

    def issue(g, b):
        gofs = pl.multiple_of(g * (GE * K), 8)
        pltpu.async_copy(
            table.at[idx_v.at[pl.ds(gofs, GE * K)]], rows_v.at[b], gsems[b])

    for b0 in range(NBUF):
        issue(b0, b0)

    lanes = lax.iota(jnp.int32, LANES)
    gdn = lax.GatherDimensionNumbers(
        offset_dims=(), collapsed_slice_dims=(0,), start_index_map=(0,))

    def ring_body(i, carry):
        for b in range(NBUF):
            g = i * NBUF + b
            gofs = pl.multiple_of(g * (GE * K), 8)
            pltpu.make_async_copy(
                table.at[idx_v.at[pl.ds(gofs, GE * K)]], rows_v.at[b],
                gsems[b]).wait()
            for ee in range(GE):
                e = g * GE + ee
                iv = idx_v[pl.ds((g * GE + ee) * K, K)]
                cnt = jnp.minimum(iv.astype(jnp.float32), 1.0)
                for s in (1, 2, 4, 8):
                    perm = jnp.bitwise_xor(lanes, s)
                    shuf = lax.gather(cnt, perm[:, None], gdn, (1,),
                                      mode=lax.GatherScatterMode.PROMISE_IN_BOUNDS)
                    cnt = cnt + shuf
                cnt1 = jnp.maximum(cnt, 1.0)
                scale = 1.0 / cnt1
                c0 = 16.0 - cnt1
                slot = lax.rem(e, OG)

                def fbody(f, fc):
                    col = pl.ds(f * LANES, LANES)
                    acc = rows_v[b, ee * K, col]
                    for k2 in range(1, K):
                        acc = acc + rows_v[b, ee * K + k2, col]
                    out_v[slot, col] = (acc - c0 * x0_v[0, col]) * scale
                    return fc

                lax.fori_loop(0, D // LANES, fbody, 0, unroll=2)

                @pl.when(slot == OG - 1)
                def _():
                    row0 = pl.multiple_of(base + e - (OG - 1), OG)
                    pltpu.sync_copy(out_v, out.at[pl.ds(row0, OG)])

            @pl.when(g + NBUF < ngrp)
            def _():
                issue(g + NBUF, b)
        return carry

    lax.fori_loop(0, ngrp // NBUF, ring_body, 0)


def _mm_body(a_ref, w1_ref, w2_ref, o_ref):
    t = jnp.dot(a_ref[...], w1_ref[...], preferred_element_type=jnp.float32)
    t = jnp.maximum(t, 0.0)
    o_ref[...] = jnp.dot(t, w2_ref[...], preferred_element_type=jnp.float32)


def _mm(a, W1, W2):
    br = 512
    return pl.pallas_call(
        _mm_body,
        grid=(BP // br,),
        in_specs=[
            pl.BlockSpec((br, D), lambda i: (i, 0)),
            pl.BlockSpec((D, D), lambda i: (0, 0)),
            pl.BlockSpec((D, D), lambda i: (0, 0)),
        ],
        out_specs=pl.BlockSpec((br, D), lambda i: (i, 0)),
        out_shape=jax.ShapeDtypeStruct((BP, D), jnp.float32),
    )(a, W1, W2)


def kernel(x, seq, useq, W1, W2, att_w, att_b):
    e_rows = seq.shape[0]
    m_rows = useq.shape[0]
    seq_p = jnp.pad(seq, ((0, BP - e_rows), (0, 0))).reshape(-1)
    useq_p = jnp.pad(useq, ((0, BP - m_rows), (0, 0))).reshape(-1)
    agg1 = _gather_mean(x, seq_p)
    h = _mm(agg1, W1, W2)
    node = _gather_mean(h, useq_p)
    return node[:m_rows]
